# Initial kernel scaffold; baseline (speedup 1.0000x reference)
import jax
import jax.numpy as jnp
from jax.experimental import pallas as pl


def kernel(x, edge_index, lane_x, lane_edge_index, focal_idx, centerline, W_ae, b_ae, a1_Wl, a1_Wr, a1_att, a1_b, a2_Wl, a2_Wr, a2_att, a2_b, W_me, b_me, W_cl, b_cl, g_Wl, g_Wr, g_att, g_b, W_fc1, b_fc1, W_fc2, b_fc2, W_out, b_out):
    raise NotImplementedError("write your pallas kernel here")


# speedup vs baseline: 15.2998x; 15.2998x over previous
import functools

import jax
import jax.numpy as jnp
from jax import lax
from jax.experimental import pallas as pl
from jax.experimental.pallas import tpu as pltpu
from jax.experimental.pallas import tpu_sc as plsc

_N = 10000
_E = 320000
_HD = 32
_HM = 128
_NP = 10240
_K = 64
_NW = 32
_CPW = 162
_NCH = _NW * _CPW
_ENP = _NCH * _K


def _perm(v, idx):
    return lax.gather(
        v, idx[:, None],
        lax.GatherDimensionNumbers(
            offset_dims=(), collapsed_slice_dims=(0,), start_index_map=(0,)),
        (1,), mode=lax.GatherScatterMode.PROMISE_IN_BOUNDS)


def _make_sc_gat(heads):
    D = heads * _HD
    RW = D + 16
    rows_per_tile = _NP // 16
    mesh = plsc.VectorSubcoreMesh(core_axis_name="c", subcore_axis_name="s")

    @functools.partial(
        pl.kernel,
        out_type=jax.ShapeDtypeStruct((2, _NP, RW), jnp.float32),
        mesh=mesh,
        scratch_types=[
            pltpu.VMEM_SHARED((_NP, RW), jnp.float32),
            pltpu.VMEM((_K, D), jnp.float32),
            pltpu.VMEM((_K, D), jnp.float32),
            pltpu.VMEM((_K, RW), jnp.float32),
            pltpu.VMEM((_K,), jnp.int32),
            pltpu.VMEM((_K,), jnp.int32),
            pltpu.VMEM((D,), jnp.float32),
            pltpu.SemaphoreType.DMA,
            pltpu.SemaphoreType.DMA,
        ],
        compiler_params=pltpu.CompilerParams(use_tc_tiling_on_sc=False),
    )
    def sc_gat(xl_hbm, xr_hbm, src_hbm, dst_hbm, att_hbm, out_hbm,
               acc, xl_buf, xr_buf, msg_buf, src_v, dst_v, att_v,
               sem1, sem2):
        c = lax.axis_index("c")
        s = lax.axis_index("s")
        lane = lax.broadcasted_iota(jnp.int32, (16,), 0)
        zv = jnp.zeros((16,), jnp.float32)

        def zrow(i, carry):
            for g in range(RW // 16):
                msg_buf[i, pl.ds(g * 16, 16)] = zv
            return carry
        lax.fori_loop(0, _K, zrow, 0)
        base_row = s * rows_per_tile

        def zcp(i, carry):
            pltpu.sync_copy(msg_buf, acc.at[pl.ds(base_row + i * _K, _K)])
            return carry
        lax.fori_loop(0, rows_per_tile // _K, zcp, 0)
        plsc.subcore_barrier()

        pltpu.sync_copy(att_hbm, att_v)
        perms = [lane ^ d for d in (8, 4, 2, 1)]
        cbase = (c * 16 + s) * _CPW

        def edge(e, carry):
            den = zv
            for h in range(heads):
                xs, ws = [], []
                for k in range(2):
                    off = h * _HD + k * 16
                    av = xl_buf[e, pl.ds(off, 16)]
                    bv = xr_buf[e, pl.ds(off, 16)]
                    sv = av + bv
                    tv = jnp.maximum(sv, 0.2 * sv)
                    ws.append(tv * att_v[pl.ds(off, 16)])
                    xs.append(av)
                g = ws[0] + ws[1]
                for p in perms:
                    g = g + _perm(g, p)
                ex = jnp.exp(g)
                msg_buf[e, pl.ds(h * _HD, 16)] = ex * xs[0]
                msg_buf[e, pl.ds(h * _HD + 16, 16)] = ex * xs[1]
                den = jnp.where(lane == h, ex, den)
            msg_buf[e, pl.ds(D, 16)] = den
            return carry

        def chunk_body(j, carry):
            pltpu.sync_copy(src_hbm.at[cbase + j], src_v)
            pltpu.sync_copy(dst_hbm.at[cbase + j], dst_v)
            pltpu.async_copy(xl_hbm.at[src_v], xl_buf, sem1).wait()
            pltpu.async_copy(xr_hbm.at[dst_v], xr_buf, sem2).wait()
            lax.fori_loop(0, _K, edge, 0)
            pltpu.sync_copy(msg_buf, acc.at[dst_v], add=True)
            return carry
        lax.fori_loop(0, _CPW, chunk_body, 0)
        plsc.subcore_barrier()
        pltpu.sync_copy(acc.at[pl.ds(base_row, rows_per_tile)],
                        out_hbm.at[c, pl.ds(base_row, rows_per_tile)])

    return sc_gat


_sc_gat4 = _make_sc_gat(4)
_sc_gat1 = _make_sc_gat(1)


def _dot(a, b):
    return jnp.dot(a, b, preferred_element_type=jnp.float32)


def _tc1_body(xf, wae, bae, wl, wr, xl_o, xr_o):
    ax = jnp.maximum(_dot(xf[...], wae[...]) + bae[...], 0.0)
    xl_o[...] = _dot(ax, wl[...])
    xr_o[...] = _dot(ax, wr[...])


def _tc1(xf, wae, bae, wl, wr):
    return pl.pallas_call(
        _tc1_body,
        out_shape=(jax.ShapeDtypeStruct((_N, _HM), jnp.float32),
                   jax.ShapeDtypeStruct((_N, _HM), jnp.float32)),
    )(xf, wae, bae, wl, wr)


def _tc_mid_body(p, b, wl, wr, xl_o, xr_o):
    m = p[0] + p[1]
    msg = m[:_N, :_HM]
    den16 = m[:_N, _HM:_HM + 16]
    i0 = lax.broadcasted_iota(jnp.int32, (16, _HM), 0)
    i1 = lax.broadcasted_iota(jnp.int32, (16, _HM), 1)
    sel = (i0 == i1 // _HD).astype(jnp.float32)
    denb = _dot(den16, sel)
    a = jnp.maximum(msg / (denb + 1e-16) + b[...], 0.0)
    xl_o[...] = _dot(a, wl[...])
    xr_o[...] = _dot(a, wr[...])


def _tc_mid(p, b, wl, wr):
    dout = wl.shape[1]
    return pl.pallas_call(
        _tc_mid_body,
        out_shape=(jax.ShapeDtypeStruct((_N, dout), jnp.float32),
                   jax.ShapeDtypeStruct((_N, dout), jnp.float32)),
    )(p, b, wl, wr)


def _tc_head_body(pf, gb, lx, wme, bme, c2, wcl, bcl,
                  wf1, bf1, wf2, bf2, wo, bo, out_o):
    i0 = lax.broadcasted_iota(jnp.int32, (16, _HD), 0)
    sel = (i0 == 0).astype(jnp.float32)
    den = _dot(pf[:, _HD:_HD + 16], sel)
    gf = jnp.maximum(pf[:, :_HD] / (den + 1e-16) + gb[...], 0.0)
    mapx = jnp.maximum(_dot(lx[...], wme[...]) + bme[...], 0.0)
    mg = jnp.sum(mapx, axis=0, keepdims=True) * (1.0 / lx.shape[0])
    a0 = lax.broadcasted_iota(jnp.int32, (100, 2), 0)
    a1 = lax.broadcasted_iota(jnp.int32, (100, 2), 1)
    avg = ((a0 % 2) == a1).astype(jnp.float32) * (1.0 / 50.0)
    cl = _dot(c2[...], _dot(avg, wcl[...])) + bcl[...]
    h1 = jnp.maximum(_dot(gf, wf1[:_HD]) + _dot(mg, wf1[_HD:2 * _HD])
                     + _dot(cl, wf1[2 * _HD:]) + bf1[...], 0.0)
    h2 = jnp.maximum(_dot(h1, wf2[...]) + bf2[...], 0.0)
    out_o[...] = _dot(h2, wo[...]) + bo[...]


def _tc_head(pf, gb, lx, wme, bme, c2, wcl, bcl, wf1, bf1, wf2, bf2, wo, bo):
    nf = pf.shape[0]
    return pl.pallas_call(
        _tc_head_body,
        out_shape=jax.ShapeDtypeStruct((nf, wo.shape[1]), jnp.float32),
    )(pf, gb, lx, wme, bme, c2, wcl, bcl, wf1, bf1, wf2, bf2, wo, bo)


def kernel(x, edge_index, lane_x, lane_edge_index, focal_idx, centerline,
           W_ae, b_ae, a1_Wl, a1_Wr, a1_att, a1_b, a2_Wl, a2_Wr, a2_att, a2_b,
           W_me, b_me, W_cl, b_cl, g_Wl, g_Wr, g_att, g_b,
           W_fc1, b_fc1, W_fc2, b_fc2, W_out, b_out):
    n = x.shape[0]
    loops = jnp.arange(n, dtype=edge_index.dtype)
    src = jnp.concatenate([edge_index[0], loops])
    dst = jnp.concatenate([edge_index[1], loops])
    pad = _ENP - src.shape[0]
    src = jnp.concatenate([src, jnp.zeros((pad,), src.dtype)])
    dst = jnp.concatenate([dst, jnp.full((pad,), n, dst.dtype)])
    src2 = src.reshape(_NCH, _K)
    dst2 = dst.reshape(_NCH, _K)

    xf = x.reshape(n, -1)
    xl1, xr1 = _tc1(xf, W_ae, b_ae.reshape(1, -1), a1_Wl, a1_Wr)
    p1 = _sc_gat4(xl1, xr1, src2, dst2, a1_att.reshape(-1))
    xl2, xr2 = _tc_mid(p1, a1_b.reshape(1, -1), a2_Wl, a2_Wr)
    p2 = _sc_gat4(xl2, xr2, src2, dst2, a2_att.reshape(-1))
    xlg, xrg = _tc_mid(p2, a2_b.reshape(1, -1), g_Wl, g_Wr)
    p3 = _sc_gat1(xlg, xrg, src2, dst2, g_att.reshape(-1))

    pf = p3[0, focal_idx] + p3[1, focal_idx]
    c2 = centerline.reshape(centerline.shape[0], -1)
    return _tc_head(pf, g_b.reshape(1, -1), lane_x, W_me, b_me.reshape(1, -1),
                    c2, W_cl, b_cl.reshape(1, -1), W_fc1, b_fc1.reshape(1, -1),
                    W_fc2, b_fc2.reshape(1, -1), W_out, b_out.reshape(1, -1))


# speedup vs baseline: 21.1985x; 1.3855x over previous
import functools

import jax
import jax.numpy as jnp
from jax import lax
from jax.experimental import pallas as pl
from jax.experimental.pallas import tpu as pltpu
from jax.experimental.pallas import tpu_sc as plsc

_N = 10000
_E = 320000
_HD = 32
_HM = 128
_NP = 10240
_K = 48
_IB = 8
_NW = 32
_CPW = 216
_NB = _CPW // _IB
_NCH = _NW * _CPW
_ENP = _NCH * _K


def _perm(v, idx):
    return lax.gather(
        v, idx[:, None],
        lax.GatherDimensionNumbers(
            offset_dims=(), collapsed_slice_dims=(0,), start_index_map=(0,)),
        (1,), mode=lax.GatherScatterMode.PROMISE_IN_BOUNDS)


def _make_sc_gat(heads):
    D = heads * _HD
    RW = D + 16
    rows_per_tile = _NP // 16
    mesh = plsc.VectorSubcoreMesh(core_axis_name="c", subcore_axis_name="s")

    @functools.partial(
        pl.kernel,
        out_type=jax.ShapeDtypeStruct((2, _NP, RW), jnp.float32),
        mesh=mesh,
        scratch_types=[
            pltpu.VMEM_SHARED((_NP, RW), jnp.float32),
            pltpu.VMEM((2, _K, D), jnp.float32),
            pltpu.VMEM((2, _K, D), jnp.float32),
            pltpu.VMEM((_K, RW), jnp.float32),
            pltpu.VMEM((_IB, _K), jnp.int32),
            pltpu.VMEM((_IB, _K), jnp.int32),
            pltpu.VMEM((D,), jnp.float32),
            pltpu.SemaphoreType.DMA,
            pltpu.SemaphoreType.DMA,
            pltpu.SemaphoreType.DMA,
            pltpu.SemaphoreType.DMA,
        ],
        compiler_params=pltpu.CompilerParams(use_tc_tiling_on_sc=False),
    )
    def sc_gat(xl_hbm, xr_hbm, src_hbm, dst_hbm, att_hbm, out_hbm,
               acc, xl_buf, xr_buf, msg_buf, src_b, dst_b, att_v,
               semla, semra, semlb, semrb):
        c = lax.axis_index("c")
        s = lax.axis_index("s")
        lane = lax.broadcasted_iota(jnp.int32, (16,), 0)
        zv = jnp.zeros((16,), jnp.float32)

        def zrow(i, carry):
            for g in range(RW // 16):
                msg_buf[i, pl.ds(g * 16, 16)] = zv
            return carry
        lax.fori_loop(0, 40, zrow, 0)
        base_row = s * rows_per_tile

        def zcp(i, carry):
            pltpu.sync_copy(msg_buf.at[pl.ds(0, 40)],
                            acc.at[pl.ds(base_row + i * 40, 40)])
            return carry
        lax.fori_loop(0, rows_per_tile // 40, zcp, 0)
        plsc.subcore_barrier()

        pltpu.sync_copy(att_hbm, att_v)
        perms = [lane ^ d for d in (8, 4, 2, 1)]
        cbase = (c * 16 + s) * _CPW

        def edge(bufsel):
            def body(e, carry):
                den = zv
                for h in range(heads):
                    xs, ws = [], []
                    for k in range(2):
                        off = h * _HD + k * 16
                        av = xl_buf[bufsel, e, pl.ds(off, 16)]
                        bv = xr_buf[bufsel, e, pl.ds(off, 16)]
                        sv = av + bv
                        tv = jnp.maximum(sv, 0.2 * sv)
                        ws.append(tv * att_v[pl.ds(off, 16)])
                        xs.append(av)
                    g = ws[0] + ws[1]
                    for p in perms:
                        g = g + _perm(g, p)
                    ex = jnp.exp(g)
                    msg_buf[e, pl.ds(h * _HD, 16)] = ex * xs[0]
                    msg_buf[e, pl.ds(h * _HD + 16, 16)] = ex * xs[1]
                    den = jnp.where(lane == h, ex, den)
                msg_buf[e, pl.ds(D, 16)] = den
                return carry
            return body

        def start_gather(i, bufsel):
            sl, sr = (semla, semra) if bufsel == 0 else (semlb, semrb)
            cl = pltpu.async_copy(xl_hbm.at[src_b.at[i]], xl_buf.at[bufsel], sl)
            cr = pltpu.async_copy(xr_hbm.at[dst_b.at[i]], xr_buf.at[bufsel], sr)
            return cl, cr

        def block_body(b, carry):
            row0 = cbase + b * _IB
            pltpu.sync_copy(src_hbm.at[pl.ds(row0, _IB)], src_b)
            pltpu.sync_copy(dst_hbm.at[pl.ds(row0, _IB)], dst_b)
            cps = start_gather(0, 0)
            for i in range(_IB):
                sel = i % 2
                if i + 1 < _IB:
                    nxt = start_gather(i + 1, 1 - sel)
                cps[0].wait()
                cps[1].wait()
                lax.fori_loop(0, _K, edge(sel), 0)
                pltpu.sync_copy(msg_buf, acc.at[dst_b.at[i]], add=True)
                if i + 1 < _IB:
                    cps = nxt
            return carry
        lax.fori_loop(0, _NB, block_body, 0)
        plsc.subcore_barrier()
        pltpu.sync_copy(acc.at[pl.ds(base_row, rows_per_tile)],
                        out_hbm.at[c, pl.ds(base_row, rows_per_tile)])

    return sc_gat


_sc_gat4 = _make_sc_gat(4)
_sc_gat1 = _make_sc_gat(1)


def _dot(a, b):
    return jnp.dot(a, b, preferred_element_type=jnp.float32)


def _tc1_body(xf, wae, bae, wl, wr, xl_o, xr_o):
    ax = jnp.maximum(_dot(xf[...], wae[...]) + bae[...], 0.0)
    xl_o[...] = _dot(ax, wl[...])
    xr_o[...] = _dot(ax, wr[...])


def _tc1(xf, wae, bae, wl, wr):
    return pl.pallas_call(
        _tc1_body,
        out_shape=(jax.ShapeDtypeStruct((_N, _HM), jnp.float32),
                   jax.ShapeDtypeStruct((_N, _HM), jnp.float32)),
    )(xf, wae, bae, wl, wr)


def _tc_mid_body(p, b, wl, wr, xl_o, xr_o):
    m = p[0] + p[1]
    msg = m[:_N, :_HM]
    den16 = m[:_N, _HM:_HM + 16]
    i0 = lax.broadcasted_iota(jnp.int32, (16, _HM), 0)
    i1 = lax.broadcasted_iota(jnp.int32, (16, _HM), 1)
    sel = (i0 == i1 // _HD).astype(jnp.float32)
    denb = _dot(den16, sel)
    a = jnp.maximum(msg / (denb + 1e-16) + b[...], 0.0)
    xl_o[...] = _dot(a, wl[...])
    xr_o[...] = _dot(a, wr[...])


def _tc_mid(p, b, wl, wr):
    dout = wl.shape[1]
    return pl.pallas_call(
        _tc_mid_body,
        out_shape=(jax.ShapeDtypeStruct((_N, dout), jnp.float32),
                   jax.ShapeDtypeStruct((_N, dout), jnp.float32)),
    )(p, b, wl, wr)


def _tc_head_body(pf, gb, lx, wme, bme, c2, wcl, bcl,
                  wf1, bf1, wf2, bf2, wo, bo, out_o):
    i0 = lax.broadcasted_iota(jnp.int32, (16, _HD), 0)
    sel = (i0 == 0).astype(jnp.float32)
    den = _dot(pf[:, _HD:_HD + 16], sel)
    gf = jnp.maximum(pf[:, :_HD] / (den + 1e-16) + gb[...], 0.0)
    mapx = jnp.maximum(_dot(lx[...], wme[...]) + bme[...], 0.0)
    mg = jnp.sum(mapx, axis=0, keepdims=True) * (1.0 / lx.shape[0])
    a0 = lax.broadcasted_iota(jnp.int32, (100, 2), 0)
    a1 = lax.broadcasted_iota(jnp.int32, (100, 2), 1)
    avg = ((a0 % 2) == a1).astype(jnp.float32) * (1.0 / 50.0)
    cl = _dot(c2[...], _dot(avg, wcl[...])) + bcl[...]
    h1 = jnp.maximum(_dot(gf, wf1[:_HD]) + _dot(mg, wf1[_HD:2 * _HD])
                     + _dot(cl, wf1[2 * _HD:]) + bf1[...], 0.0)
    h2 = jnp.maximum(_dot(h1, wf2[...]) + bf2[...], 0.0)
    out_o[...] = _dot(h2, wo[...]) + bo[...]


def _tc_head(pf, gb, lx, wme, bme, c2, wcl, bcl, wf1, bf1, wf2, bf2, wo, bo):
    nf = pf.shape[0]
    return pl.pallas_call(
        _tc_head_body,
        out_shape=jax.ShapeDtypeStruct((nf, wo.shape[1]), jnp.float32),
    )(pf, gb, lx, wme, bme, c2, wcl, bcl, wf1, bf1, wf2, bf2, wo, bo)


def kernel(x, edge_index, lane_x, lane_edge_index, focal_idx, centerline,
           W_ae, b_ae, a1_Wl, a1_Wr, a1_att, a1_b, a2_Wl, a2_Wr, a2_att, a2_b,
           W_me, b_me, W_cl, b_cl, g_Wl, g_Wr, g_att, g_b,
           W_fc1, b_fc1, W_fc2, b_fc2, W_out, b_out):
    n = x.shape[0]
    loops = jnp.arange(n, dtype=edge_index.dtype)
    src = jnp.concatenate([edge_index[0], loops])
    dst = jnp.concatenate([edge_index[1], loops])
    pad = _ENP - src.shape[0]
    src = jnp.concatenate([src, jnp.zeros((pad,), src.dtype)])
    dst = jnp.concatenate([dst, jnp.full((pad,), n, dst.dtype)])
    src2 = src.reshape(_NCH, _K)
    dst2 = dst.reshape(_NCH, _K)

    xf = x.reshape(n, -1)
    xl1, xr1 = _tc1(xf, W_ae, b_ae.reshape(1, -1), a1_Wl, a1_Wr)
    p1 = _sc_gat4(xl1, xr1, src2, dst2, a1_att.reshape(-1))
    xl2, xr2 = _tc_mid(p1, a1_b.reshape(1, -1), a2_Wl, a2_Wr)
    p2 = _sc_gat4(xl2, xr2, src2, dst2, a2_att.reshape(-1))
    xlg, xrg = _tc_mid(p2, a2_b.reshape(1, -1), g_Wl, g_Wr)
    p3 = _sc_gat1(xlg, xrg, src2, dst2, g_att.reshape(-1))

    pf = p3[0, focal_idx] + p3[1, focal_idx]
    c2 = centerline.reshape(centerline.shape[0], -1)
    return _tc_head(pf, g_b.reshape(1, -1), lane_x, W_me, b_me.reshape(1, -1),
                    c2, W_cl, b_cl.reshape(1, -1), W_fc1, b_fc1.reshape(1, -1),
                    W_fc2, b_fc2.reshape(1, -1), W_out, b_out.reshape(1, -1))


# speedup vs baseline: 57.5368x; 2.7142x over previous
import functools

import jax
import jax.numpy as jnp
from jax import lax
from jax.experimental import pallas as pl
from jax.experimental.pallas import tpu as pltpu
from jax.experimental.pallas import tpu_sc as plsc

_N = 10000
_E = 320000
_HD = 32
_HM = 128
_NP = 10240
_K = 48
_IB = 8
_NW = 32
_CPW = 216
_NB = _CPW // _IB
_NCH = _NW * _CPW
_ENP = _NCH * _K


def _perm(v, idx):
    return lax.gather(
        v, idx[:, None],
        lax.GatherDimensionNumbers(
            offset_dims=(), collapsed_slice_dims=(0,), start_index_map=(0,)),
        (1,), mode=lax.GatherScatterMode.PROMISE_IN_BOUNDS)


def _make_sc_gat(heads):
    D = heads * _HD
    RW = D + 16
    rows_per_tile = _NP // 16
    mesh = plsc.VectorSubcoreMesh(core_axis_name="c", subcore_axis_name="s")

    @functools.partial(
        pl.kernel,
        out_type=jax.ShapeDtypeStruct((2, _NP, RW), jnp.float32),
        mesh=mesh,
        scratch_types=[
            pltpu.VMEM_SHARED((_NP, RW), jnp.float32),
            pltpu.VMEM((2, _K, D), jnp.float32),
            pltpu.VMEM((2, _K, D), jnp.float32),
            pltpu.VMEM((_K, RW), jnp.float32),
            pltpu.VMEM((_IB, _K), jnp.int32),
            pltpu.VMEM((_IB, _K), jnp.int32),
            pltpu.VMEM((D,), jnp.float32),
            pltpu.SemaphoreType.DMA,
            pltpu.SemaphoreType.DMA,
            pltpu.SemaphoreType.DMA,
            pltpu.SemaphoreType.DMA,
        ],
        compiler_params=pltpu.CompilerParams(use_tc_tiling_on_sc=False),
    )
    def sc_gat(xl_hbm, xr_hbm, src_hbm, dst_hbm, att_hbm, out_hbm,
               acc, xl_buf, xr_buf, msg_buf, src_b, dst_b, att_v,
               semla, semra, semlb, semrb):
        c = lax.axis_index("c")
        s = lax.axis_index("s")
        lane = lax.broadcasted_iota(jnp.int32, (16,), 0)
        zv = jnp.zeros((16,), jnp.float32)

        def zrow(i, carry):
            for g in range(RW // 16):
                msg_buf[i, pl.ds(g * 16, 16)] = zv
            return carry
        lax.fori_loop(0, 40, zrow, 0)
        base_row = s * rows_per_tile

        def zcp(i, carry):
            pltpu.sync_copy(msg_buf.at[pl.ds(0, 40)],
                            acc.at[pl.ds(base_row + i * 40, 40)])
            return carry
        lax.fori_loop(0, rows_per_tile // 40, zcp, 0)
        plsc.subcore_barrier()

        pltpu.sync_copy(att_hbm, att_v)
        perms = [lane ^ d for d in (8, 4, 2, 1)]
        cbase = (c * 16 + s) * _CPW

        def edge(bufsel):
            def body(e):
                den = zv
                for h in range(heads):
                    xs, ws = [], []
                    for k in range(2):
                        off = h * _HD + k * 16
                        av = xl_buf[bufsel, e, pl.ds(off, 16)]
                        bv = xr_buf[bufsel, e, pl.ds(off, 16)]
                        sv = av + bv
                        tv = jnp.maximum(sv, 0.2 * sv)
                        ws.append(tv * att_v[pl.ds(off, 16)])
                        xs.append(av)
                    g = ws[0] + ws[1]
                    for p in perms:
                        g = g + _perm(g, p)
                    ex = jnp.exp(g)
                    msg_buf[e, pl.ds(h * _HD, 16)] = ex * xs[0]
                    msg_buf[e, pl.ds(h * _HD + 16, 16)] = ex * xs[1]
                    den = jnp.where(lane == h, ex, den)
                msg_buf[e, pl.ds(D, 16)] = den
            return body

        def start_gather(i, bufsel):
            sl, sr = (semla, semra) if bufsel == 0 else (semlb, semrb)
            cl = pltpu.async_copy(xl_hbm.at[src_b.at[i]], xl_buf.at[bufsel], sl)
            cr = pltpu.async_copy(xr_hbm.at[dst_b.at[i]], xr_buf.at[bufsel], sr)
            return cl, cr

        def block_body(b, carry):
            row0 = cbase + b * _IB
            pltpu.sync_copy(src_hbm.at[pl.ds(row0, _IB)], src_b)
            pltpu.sync_copy(dst_hbm.at[pl.ds(row0, _IB)], dst_b)
            cps = start_gather(0, 0)
            for i in range(_IB):
                sel = i % 2
                if i + 1 < _IB:
                    nxt = start_gather(i + 1, 1 - sel)
                cps[0].wait()
                cps[1].wait()
                plsc.parallel_loop(0, _K, unroll=2)(edge(sel))
                pltpu.sync_copy(msg_buf, acc.at[dst_b.at[i]], add=True)
                if i + 1 < _IB:
                    cps = nxt
            return carry
        lax.fori_loop(0, _NB, block_body, 0)
        plsc.subcore_barrier()
        pltpu.sync_copy(acc.at[pl.ds(base_row, rows_per_tile)],
                        out_hbm.at[c, pl.ds(base_row, rows_per_tile)])

    return sc_gat


_sc_gat4 = _make_sc_gat(4)
_sc_gat1 = _make_sc_gat(1)


def _dot(a, b):
    return jnp.dot(a, b, preferred_element_type=jnp.float32)


def _tc1_body(xf, wae, bae, wl, wr, xl_o, xr_o):
    ax = jnp.maximum(_dot(xf[...], wae[...]) + bae[...], 0.0)
    xl_o[...] = _dot(ax, wl[...])
    xr_o[...] = _dot(ax, wr[...])


def _tc1(xf, wae, bae, wl, wr):
    return pl.pallas_call(
        _tc1_body,
        out_shape=(jax.ShapeDtypeStruct((_N, _HM), jnp.float32),
                   jax.ShapeDtypeStruct((_N, _HM), jnp.float32)),
    )(xf, wae, bae, wl, wr)


def _tc_mid_body(p, b, wl, wr, xl_o, xr_o):
    m = p[0] + p[1]
    msg = m[:_N, :_HM]
    den16 = m[:_N, _HM:_HM + 16]
    i0 = lax.broadcasted_iota(jnp.int32, (16, _HM), 0)
    i1 = lax.broadcasted_iota(jnp.int32, (16, _HM), 1)
    sel = (i0 == i1 // _HD).astype(jnp.float32)
    denb = _dot(den16, sel)
    a = jnp.maximum(msg / (denb + 1e-16) + b[...], 0.0)
    xl_o[...] = _dot(a, wl[...])
    xr_o[...] = _dot(a, wr[...])


def _tc_mid(p, b, wl, wr):
    dout = wl.shape[1]
    return pl.pallas_call(
        _tc_mid_body,
        out_shape=(jax.ShapeDtypeStruct((_N, dout), jnp.float32),
                   jax.ShapeDtypeStruct((_N, dout), jnp.float32)),
    )(p, b, wl, wr)


def _tc_head_body(pf, gb, lx, wme, bme, c2, wcl, bcl,
                  wf1, bf1, wf2, bf2, wo, bo, out_o):
    i0 = lax.broadcasted_iota(jnp.int32, (16, _HD), 0)
    sel = (i0 == 0).astype(jnp.float32)
    den = _dot(pf[:, _HD:_HD + 16], sel)
    gf = jnp.maximum(pf[:, :_HD] / (den + 1e-16) + gb[...], 0.0)
    mapx = jnp.maximum(_dot(lx[...], wme[...]) + bme[...], 0.0)
    mg = jnp.sum(mapx, axis=0, keepdims=True) * (1.0 / lx.shape[0])
    a0 = lax.broadcasted_iota(jnp.int32, (100, 2), 0)
    a1 = lax.broadcasted_iota(jnp.int32, (100, 2), 1)
    avg = ((a0 % 2) == a1).astype(jnp.float32) * (1.0 / 50.0)
    cl = _dot(c2[...], _dot(avg, wcl[...])) + bcl[...]
    h1 = jnp.maximum(_dot(gf, wf1[:_HD]) + _dot(mg, wf1[_HD:2 * _HD])
                     + _dot(cl, wf1[2 * _HD:]) + bf1[...], 0.0)
    h2 = jnp.maximum(_dot(h1, wf2[...]) + bf2[...], 0.0)
    out_o[...] = _dot(h2, wo[...]) + bo[...]


def _tc_head(pf, gb, lx, wme, bme, c2, wcl, bcl, wf1, bf1, wf2, bf2, wo, bo):
    nf = pf.shape[0]
    return pl.pallas_call(
        _tc_head_body,
        out_shape=jax.ShapeDtypeStruct((nf, wo.shape[1]), jnp.float32),
    )(pf, gb, lx, wme, bme, c2, wcl, bcl, wf1, bf1, wf2, bf2, wo, bo)


def kernel(x, edge_index, lane_x, lane_edge_index, focal_idx, centerline,
           W_ae, b_ae, a1_Wl, a1_Wr, a1_att, a1_b, a2_Wl, a2_Wr, a2_att, a2_b,
           W_me, b_me, W_cl, b_cl, g_Wl, g_Wr, g_att, g_b,
           W_fc1, b_fc1, W_fc2, b_fc2, W_out, b_out):
    n = x.shape[0]
    loops = jnp.arange(n, dtype=edge_index.dtype)
    src = jnp.concatenate([edge_index[0], loops])
    dst = jnp.concatenate([edge_index[1], loops])
    pad = _ENP - src.shape[0]
    src = jnp.concatenate([src, jnp.zeros((pad,), src.dtype)])
    dst = jnp.concatenate([dst, jnp.full((pad,), n, dst.dtype)])
    src2 = src.reshape(_NCH, _K)
    dst2 = dst.reshape(_NCH, _K)

    xf = x.reshape(n, -1)
    xl1, xr1 = _tc1(xf, W_ae, b_ae.reshape(1, -1), a1_Wl, a1_Wr)
    p1 = _sc_gat4(xl1, xr1, src2, dst2, a1_att.reshape(-1))
    xl2, xr2 = _tc_mid(p1, a1_b.reshape(1, -1), a2_Wl, a2_Wr)
    p2 = _sc_gat4(xl2, xr2, src2, dst2, a2_att.reshape(-1))
    xlg, xrg = _tc_mid(p2, a2_b.reshape(1, -1), g_Wl, g_Wr)
    p3 = _sc_gat1(xlg, xrg, src2, dst2, g_att.reshape(-1))

    pf = p3[0, focal_idx] + p3[1, focal_idx]
    c2 = centerline.reshape(centerline.shape[0], -1)
    return _tc_head(pf, g_b.reshape(1, -1), lane_x, W_me, b_me.reshape(1, -1),
                    c2, W_cl, b_cl.reshape(1, -1), W_fc1, b_fc1.reshape(1, -1),
                    W_fc2, b_fc2.reshape(1, -1), W_out, b_out.reshape(1, -1))
